# baseline (speedup 1.0000x reference)
import functools

import jax
import jax.numpy as jnp
from jax import lax
from jax.experimental import pallas as pl
from jax.experimental.pallas import tpu as pltpu
from jax.experimental.pallas import tpu_sc as plsc

_D = 64
_M = 2145
_MP = 2152
_B = 8192
_NC = 2
_NS = 16
_NW = _NC * _NS
_BW = _B // _NW
_NK = _BW // 16
_BLK = 16
_NBLK = 134


def _make_poly():
    mesh = plsc.VectorSubcoreMesh(core_axis_name="c", subcore_axis_name="s")

    @functools.partial(
        pl.kernel,
        mesh=mesh,
        out_type=jax.ShapeDtypeStruct((_M, _B), jnp.float32),
        compiler_params=pltpu.CompilerParams(needs_layout_passes=False),
        scratch_types=[
            pltpu.VMEM((_D + 1, _BW), jnp.float32),
            pltpu.VMEM((_MP,), jnp.int32),
            pltpu.VMEM((_MP,), jnp.int32),
            pltpu.VMEM((_BLK, _BW), jnp.float32),
            pltpu.VMEM((_BLK, _BW), jnp.float32),
            pltpu.SemaphoreType.DMA,
            pltpu.SemaphoreType.DMA,
        ],
    )
    def _poly(xt_hbm, ti_hbm, tj_hbm, out_hbm,
              x_v, ti_v, tj_v, o0, o1, s0, s1):
        wid = lax.axis_index("s") * _NC + lax.axis_index("c")
        b0 = wid * _BW
        pltpu.sync_copy(ti_hbm, ti_v)
        pltpu.sync_copy(tj_hbm, tj_v)
        pltpu.sync_copy(xt_hbm.at[:, pl.ds(b0, _BW)], x_v.at[pl.ds(0, _D), :])
        ones16 = jnp.full((16,), 1.0, jnp.float32)
        for k in range(_NK):
            x_v[_D, pl.ds(k * 16, 16)] = ones16
        os_ = (o0, o1)
        sems = (s0, s1)
        lane = lax.iota(jnp.int32, 16)

        def tile_row(mt, buf, r0):
            ivs, jvs = [], []
            for m_r in range(8):
                mvec = jnp.full((16,), jnp.int32(0), jnp.int32) + (mt * 8 + m_r)
                ivs.append(plsc.load_gather(ti_v, [mvec]))
                jvs.append(plsc.load_gather(tj_v, [mvec]))

            @plsc.parallel_loop(0, _NK, unroll=2)
            def _(k):
                colv = lane + k * 16
                for m_r in range(8):
                    a = plsc.load_gather(x_v, [ivs[m_r], colv])
                    b = plsc.load_gather(x_v, [jvs[m_r], colv])
                    buf[r0 + m_r, pl.ds(k * 16, 16)] = a * b

        def compute(j, buf):
            for t in range(_BLK // 8):
                tile_row(j * (_BLK // 8) + t, buf, t * 8)

        def start_out(j, buf, sem):
            pltpu.async_copy(buf, out_hbm.at[pl.ds(j * _BLK, _BLK),
                                             pl.ds(b0, _BW)], sem)

        def wait_out(buf, sem):
            pltpu.make_async_copy(buf, out_hbm.at[pl.ds(0, _BLK),
                                                  pl.ds(0, _BW)], sem).wait()

        for p in range(2):
            compute(jnp.int32(p), os_[p])
            if p == 0:
                for k in range(_NK):
                    os_[p][0, pl.ds(k * 16, 16)] = ones16
            start_out(jnp.int32(p), os_[p], sems[p])

        def pair_body(k, _):
            for p in range(2):
                j = 2 * k + p
                wait_out(os_[p], sems[p])
                compute(j, os_[p])
                start_out(j, os_[p], sems[p])
            return 0

        lax.fori_loop(1, _NBLK // 2, pair_body, 0)

        wait_out(os_[0], sems[0])
        compute(jnp.int32(_NBLK - 1), os_[0])
        start_out(jnp.int32(_NBLK - 1), os_[0], sems[0])
        wait_out(os_[1], sems[1])
        tile_row(jnp.int32(_M // 8), os_[1], 0)
        pltpu.async_copy(os_[1].at[pl.ds(0, 1), :],
                         out_hbm.at[pl.ds(_M - 1, 1), pl.ds(b0, _BW)], s1)
        wait_out(os_[0], sems[0])
        pltpu.make_async_copy(os_[1].at[pl.ds(0, 1), :],
                              out_hbm.at[pl.ds(0, 1), pl.ds(0, _BW)],
                              s1).wait()

    return _poly


_poly_call = _make_poly()


def kernel(x, indices):
    xt = x.T.astype(jnp.float32)
    pad = _MP - _M
    ti = jnp.pad(indices[:, 0].astype(jnp.int32), (0, pad))
    tj = jnp.pad(indices[:, 1].astype(jnp.int32), (0, pad))
    out = _poly_call(xt, ti, tj)
    return out.T


# speedup vs baseline: 1.2881x; 1.2881x over previous
import functools

import jax
import jax.numpy as jnp
from jax import lax
from jax.experimental import pallas as pl
from jax.experimental.pallas import tpu as pltpu
from jax.experimental.pallas import tpu_sc as plsc

_D = 64
_M = 2145
_MP = 2152
_B = 8192
_NC = 2
_NS = 16
_NW = _NC * _NS
_BW = _B // _NW
_NK = _BW // 16
_BLK = 8
_NBLK = 268


def _make_poly():
    mesh = plsc.VectorSubcoreMesh(core_axis_name="c", subcore_axis_name="s")

    @functools.partial(
        pl.kernel,
        mesh=mesh,
        out_type=jax.ShapeDtypeStruct((_M, _B), jnp.float32),
        compiler_params=pltpu.CompilerParams(needs_layout_passes=False),
        scratch_types=[
            pltpu.VMEM(((_D + 1) * _BW,), jnp.float32),
            pltpu.VMEM((_MP,), jnp.int32),
            pltpu.VMEM((_MP,), jnp.int32),
            pltpu.VMEM((_BLK, _BW), jnp.float32),
            pltpu.VMEM((_BLK, _BW), jnp.float32),
            pltpu.SemaphoreType.DMA,
            pltpu.SemaphoreType.DMA,
            pltpu.SemaphoreType.DMA,
        ],
    )
    def _poly(xt_hbm, ti_hbm, tj_hbm, out_hbm,
              x_v, ti_v, tj_v, o0, o1, s0, s1, sx):
        wid = lax.axis_index("s") * _NC + lax.axis_index("c")
        b0 = wid * _BW
        pltpu.sync_copy(ti_hbm, ti_v)
        pltpu.sync_copy(tj_hbm, tj_v)
        for d in range(_D):
            pltpu.async_copy(xt_hbm.at[d, pl.ds(b0, _BW)],
                             x_v.at[pl.ds(d * _BW, _BW)], sx)
        for d in range(_D):
            pltpu.make_async_copy(xt_hbm.at[d, pl.ds(0, _BW)],
                                  x_v.at[pl.ds(0, _BW)], sx).wait()
        ones16 = jnp.full((16,), 1.0, jnp.float32)
        for k in range(_NK):
            x_v[pl.ds(_D * _BW + k * 16, 16)] = ones16
        os_ = (o0, o1)
        sems = (s0, s1)
        lane = lax.iota(jnp.int32, 16)

        def tile_row(mt, buf, r0):
            fiv, fjv = [], []
            for m_r in range(8):
                mvec = jnp.full((16,), jnp.int32(0), jnp.int32) + (mt * 8 + m_r)
                iv = plsc.load_gather(ti_v, [mvec])
                jv = plsc.load_gather(tj_v, [mvec])
                fiv.append(iv * _BW + lane)
                fjv.append(jv * _BW + lane)

            @plsc.parallel_loop(0, _NK, unroll=2)
            def _(k):
                k16 = k * 16
                for m_r in range(8):
                    a = plsc.load_gather(x_v, [fiv[m_r] + k16])
                    b = plsc.load_gather(x_v, [fjv[m_r] + k16])
                    buf[r0 + m_r, pl.ds(k16, 16)] = a * b

        def compute(j, buf):
            for t in range(_BLK // 8):
                tile_row(j * (_BLK // 8) + t, buf, t * 8)

        def start_out(j, buf, sem):
            pltpu.async_copy(buf, out_hbm.at[pl.ds(j * _BLK, _BLK),
                                             pl.ds(b0, _BW)], sem)

        def wait_out(buf, sem):
            pltpu.make_async_copy(buf, out_hbm.at[pl.ds(0, _BLK),
                                                  pl.ds(0, _BW)], sem).wait()

        for p in range(2):
            compute(jnp.int32(p), os_[p])
            if p == 0:
                for k in range(_NK):
                    os_[p][0, pl.ds(k * 16, 16)] = ones16
            start_out(jnp.int32(p), os_[p], sems[p])

        def pair_body(k, _):
            for p in range(2):
                j = 2 * k + p
                wait_out(os_[p], sems[p])
                compute(j, os_[p])
                start_out(j, os_[p], sems[p])
            return 0

        lax.fori_loop(1, _NBLK // 2, pair_body, 0)

        wait_out(os_[0], sems[0])
        compute(jnp.int32(_NBLK - 1), os_[0])
        start_out(jnp.int32(_NBLK - 1), os_[0], sems[0])
        wait_out(os_[1], sems[1])
        tile_row(jnp.int32(_M // 8), os_[1], 0)
        pltpu.async_copy(os_[1].at[pl.ds(0, 1), :],
                         out_hbm.at[pl.ds(_M - 1, 1), pl.ds(b0, _BW)], s1)
        wait_out(os_[0], sems[0])
        pltpu.make_async_copy(os_[1].at[pl.ds(0, 1), :],
                              out_hbm.at[pl.ds(0, 1), pl.ds(0, _BW)],
                              s1).wait()

    return _poly


_poly_call = _make_poly()


def kernel(x, indices):
    xt = x.T.astype(jnp.float32)
    pad = _MP - _M
    ti = jnp.pad(indices[:, 0].astype(jnp.int32), (0, pad))
    tj = jnp.pad(indices[:, 1].astype(jnp.int32), (0, pad))
    out = _poly_call(xt, ti, tj)
    return out.T
